# Initial kernel scaffold; baseline (speedup 1.0000x reference)
import jax
import jax.numpy as jnp
from jax.experimental import pallas as pl


def kernel(x, edge_index, W1, b1, W2, b2, W3, b3):
    raise NotImplementedError("write your pallas kernel here")


# speedup vs baseline: 49.3629x; 49.3629x over previous
import functools

import jax
import jax.numpy as jnp
from jax import lax
from jax.experimental import pallas as pl
from jax.experimental.pallas import tpu as pltpu
from jax.experimental.pallas import tpu_sc as plsc

_NC = 2
_NS = 16
_NW = _NC * _NS
_F = 16
_CH = 8


def _mesh():
    return plsc.VectorSubcoreMesh(
        core_axis_name="c", subcore_axis_name="s",
        num_cores=_NC, num_subcores=_NS)


def _make_edge_pass(n_nodes, n_rows, gather):
    n_chunks = n_rows // _CH
    iters = pl.cdiv(n_chunks, _NW)
    n_pad = ((n_nodes + _NS * 8 - 1) // (_NS * 8)) * (_NS * 8)
    rows_per_tile = n_pad // _NS
    zrows = rows_per_tile // _NS

    scratch = [
        pltpu.VMEM((_CH, 128), jnp.int32),
        pltpu.VMEM((zrows, _F), jnp.float32),
        pltpu.VMEM_SHARED((n_pad, _F), jnp.float32),
        pltpu.SemaphoreType.DMA,
    ]
    if gather:
        scratch = [
            pltpu.VMEM((_CH, 128), jnp.int32),
            pltpu.VMEM((_CH, 128, _F), jnp.float32),
        ] + scratch

    def body(*refs):
        if gather:
            (table_hbm, src_hbm, dst_hbm, out_hbm,
             idx_s, rows, idx_d, zbuf, accum, sem) = refs
        else:
            (dst_hbm, out_hbm, idx_d, zbuf, accum, sem) = refs
        cid = lax.axis_index("c")
        sid = lax.axis_index("s")
        wid = sid * _NC + cid

        def _zero(i, _):
            zbuf[i] = jnp.zeros((_F,), jnp.float32)
            return 0
        lax.fori_loop(0, zrows, _zero, 0)
        for k in range(_NS):
            pltpu.sync_copy(
                zbuf, accum.at[pl.ds(sid * rows_per_tile + k * zrows, zrows)])
        if not gather:
            def _ones(i, _):
                zbuf[i] = jnp.ones((_F,), jnp.float32)
                return 0
            lax.fori_loop(0, 128, _ones, 0)
        plsc.subcore_barrier()

        def chunk_body(i, _):
            c = i * _NW + wid

            @pl.when(c < n_chunks)
            def _():
                r0 = c * _CH
                pltpu.sync_copy(dst_hbm.at[pl.ds(r0, _CH)], idx_d)
                if gather:
                    pltpu.sync_copy(src_hbm.at[pl.ds(r0, _CH)], idx_s)
                    cps = [pltpu.async_copy(table_hbm.at[idx_s.at[j]],
                                            rows.at[j], sem)
                           for j in range(_CH)]
                    for cp in cps:
                        cp.wait()
                    for j in range(_CH):
                        pltpu.sync_copy(rows.at[j], accum.at[idx_d.at[j]],
                                        add=True)
                else:
                    for j in range(_CH):
                        pltpu.sync_copy(zbuf.at[pl.ds(0, 128)],
                                        accum.at[idx_d.at[j]], add=True)
            return 0

        lax.fori_loop(0, iters, chunk_body, 0)
        plsc.subcore_barrier()

        off = sid * rows_per_tile
        pltpu.sync_copy(accum.at[pl.ds(off, rows_per_tile)],
                        out_hbm.at[cid, pl.ds(off, rows_per_tile)])

    return pl.kernel(
        body,
        out_type=jax.ShapeDtypeStruct((_NC, n_pad, _F), jnp.float32),
        mesh=_mesh(),
        scratch_types=scratch,
        compiler_params=pltpu.CompilerParams(use_tc_tiling_on_sc=False),
    )


def _tc(body, grid, in_specs, out_specs, out_shape):
    return pl.pallas_call(
        body, grid=grid, in_specs=in_specs, out_specs=out_specs,
        out_shape=out_shape)


def kernel(x, edge_index, W1, b1, W2, b2, W3, b3):
    n = x.shape[0]
    e = edge_index.shape[1]
    src2d = edge_index[0].reshape(-1, 128)
    dst2d = edge_index[1].reshape(-1, 128)
    n_rows = e // 128

    edge_pass = _make_edge_pass(n, n_rows, gather=True)
    deg_pass = _make_edge_pass(n, n_rows, gather=False)

    bn = 2000
    grid = (n // bn,)
    part_spec = pl.BlockSpec((_NC, bn, _F), lambda i: (0, i, 0))
    vec_spec = pl.BlockSpec((bn, 1), lambda i: (i, 0))
    f16_spec = pl.BlockSpec((bn, _F), lambda i: (i, 0))

    def full(a):
        return pl.BlockSpec(a.shape, lambda i: tuple(0 for _ in a.shape))

    def deg_body(p_ref, dis_ref):
        p = p_ref[...]
        cnt = p[0, :, 0:1] + p[1, :, 0:1]
        dis_ref[...] = lax.rsqrt(cnt + 1.0)

    def pre1_body(x_ref, w_ref, dis_ref, g_ref):
        t = jnp.dot(x_ref[...], w_ref[...],
                    preferred_element_type=jnp.float32)
        g_ref[...] = t * dis_ref[...]

    def mid1_body(p_ref, g1_ref, dis_ref, b1_ref, w2_ref, ga_ref, gb_ref):
        p = p_ref[...]
        dis = dis_ref[...]
        s = (p[0] + p[1] + g1_ref[...]) * dis + b1_ref[...]
        h = jnp.maximum(s, 0.0)
        t = jnp.dot(h, w2_ref[...], preferred_element_type=jnp.float32) * dis
        ga_ref[...] = t[:, :_F]
        gb_ref[...] = t[:, _F:]

    def mid2_body(pa_ref, pb_ref, ga_ref, gb_ref, dis_ref, b2_ref, w3_ref,
                  g3_ref):
        pa = pa_ref[...]
        pb = pb_ref[...]
        dis = dis_ref[...]
        sa = pa[0] + pa[1] + ga_ref[...]
        sb = pb[0] + pb[1] + gb_ref[...]
        s = jnp.concatenate([sa, sb], axis=1) * dis + b2_ref[...]
        h = jnp.maximum(s, 0.0)
        t = jnp.dot(h, w3_ref[...], preferred_element_type=jnp.float32) * dis
        g3_ref[...] = jnp.concatenate(
            [t, jnp.zeros((t.shape[0], _F - 1), jnp.float32)], axis=1)

    def fin_body(p_ref, g3_ref, dis_ref, b3_ref, out_ref):
        p = p_ref[...]
        s = p[0, :, 0:1] + p[1, :, 0:1] + g3_ref[...][:, 0:1]
        out_ref[...] = s * dis_ref[...] + b3_ref[...]

    b1r = b1.reshape(1, -1)
    b2r = b2.reshape(1, -1)
    b3r = b3.reshape(1, -1)

    deg_part = deg_pass(dst2d)
    dis = _tc(deg_body, grid, [part_spec], vec_spec,
              jax.ShapeDtypeStruct((n, 1), jnp.float32))(deg_part)

    g1 = _tc(pre1_body, grid,
             [pl.BlockSpec((bn, 2), lambda i: (i, 0)), full(W1), vec_spec],
             f16_spec, jax.ShapeDtypeStruct((n, _F), jnp.float32))(
                 x, W1, dis)
    p1 = edge_pass(g1, src2d, dst2d)

    g2a, g2b = _tc(mid1_body, grid,
                   [part_spec, f16_spec, vec_spec, full(b1r), full(W2)],
                   [f16_spec, f16_spec],
                   [jax.ShapeDtypeStruct((n, _F), jnp.float32)] * 2)(
                       p1, g1, dis, b1r, W2)
    p2a = edge_pass(g2a, src2d, dst2d)
    p2b = edge_pass(g2b, src2d, dst2d)

    g3 = _tc(mid2_body, grid,
             [part_spec, part_spec, f16_spec, f16_spec, vec_spec,
              full(b2r), full(W3)],
             f16_spec, jax.ShapeDtypeStruct((n, _F), jnp.float32))(
                 p2a, p2b, g2a, g2b, dis, b2r, W3)
    p3 = edge_pass(g3, src2d, dst2d)

    out = _tc(fin_body, grid,
              [part_spec, f16_spec, vec_spec, full(b3r)],
              vec_spec, jax.ShapeDtypeStruct((n, 1), jnp.float32))(
                  p3, g3, dis, b3r)
    return out


# speedup vs baseline: 60.0409x; 1.2163x over previous
import functools

import jax
import jax.numpy as jnp
from jax import lax
from jax.experimental import pallas as pl
from jax.experimental.pallas import tpu as pltpu
from jax.experimental.pallas import tpu_sc as plsc

_NC = 2
_NS = 16
_NW = _NC * _NS
_CH = 5


def _mesh():
    return plsc.VectorSubcoreMesh(
        core_axis_name="c", subcore_axis_name="s",
        num_cores=_NC, num_subcores=_NS)


def _make_edge_pass(n_nodes, n_rows, feat, gather):
    n_chunks = n_rows // _CH
    base, rem = divmod(n_chunks, _NW)
    max_steps = base + (1 if rem else 0)
    n_pad = ((n_nodes + _NS * 8 - 1) // (_NS * 8)) * (_NS * 8)
    rows_per_tile = n_pad // _NS

    scratch = {
        "idx_d0": pltpu.VMEM((_CH, 128), jnp.int32),
        "idx_d1": pltpu.VMEM((_CH, 128), jnp.int32),
        "accum": pltpu.VMEM_SHARED((n_pad, feat), jnp.float32),
    }
    if gather:
        scratch.update({
            "idx_s0": pltpu.VMEM((_CH, 128), jnp.int32),
            "idx_s1": pltpu.VMEM((_CH, 128), jnp.int32),
            "rows0": pltpu.VMEM((_CH, 128, feat), jnp.float32),
            "rows1": pltpu.VMEM((_CH, 128, feat), jnp.float32),
            "gsem0": pltpu.SemaphoreType.DMA,
            "gsem1": pltpu.SemaphoreType.DMA,
        })
    else:
        scratch["ones_v"] = pltpu.VMEM((128, feat), jnp.float32)

    def body(*refs, **kw):
        if gather:
            table_hbm, src_hbm, dst_hbm, zrow_hbm, out_hbm = refs
        else:
            dst_hbm, ones_hbm, zrow_hbm, out_hbm = refs
        cid = lax.axis_index("c")
        sid = lax.axis_index("s")
        wid = sid * _NC + cid
        accum = kw["accum"]

        pltpu.sync_copy(zrow_hbm, accum.at[pl.ds(sid * rows_per_tile,
                                                 rows_per_tile)])
        if not gather:
            pltpu.sync_copy(ones_hbm, kw["ones_v"])
        plsc.subcore_barrier()

        cnt = base + jnp.where(wid < rem, 1, 0)
        start = wid * base + jnp.minimum(wid, rem)

        if gather:
            idx_s = (kw["idx_s0"], kw["idx_s1"])
            idx_d = (kw["idx_d0"], kw["idx_d1"])
            rows = (kw["rows0"], kw["rows1"])
            gsem = (kw["gsem0"], kw["gsem1"])

            def load_idx(i, b):
                r0 = (start + i) * _CH
                pltpu.sync_copy(src_hbm.at[pl.ds(r0, _CH)], idx_s[b])
                pltpu.sync_copy(dst_hbm.at[pl.ds(r0, _CH)], idx_d[b])

            def fire(b):
                for j in range(_CH):
                    pltpu.async_copy(table_hbm.at[idx_s[b].at[j]],
                                     rows[b].at[j], gsem[b])

            def drain(b):
                for j in range(_CH):
                    pltpu.make_async_copy(table_hbm.at[idx_s[b].at[j]],
                                          rows[b].at[j], gsem[b]).wait()

            def scatter(b):
                for j in range(_CH):
                    pltpu.sync_copy(rows[b].at[j], accum.at[idx_d[b].at[j]],
                                    add=True)

            load_idx(0, 0)
            fire(0)
            load_idx(1, 1)

            def step(i, b):
                valid = i < cnt

                @pl.when(valid)
                def _():
                    drain(b)

                @pl.when(i + 1 < cnt)
                def _():
                    fire(1 - b)

                @pl.when(valid)
                def _():
                    scatter(b)

                @pl.when(i + 2 < cnt)
                def _():
                    load_idx(i + 2, b)

            def pair(j, _):
                step(2 * j, 0)
                step(2 * j + 1, 1)
                return 0

            lax.fori_loop(0, (max_steps + 2) // 2, pair, 0)
        else:
            idx_d = (kw["idx_d0"], kw["idx_d1"])
            ones_v = kw["ones_v"]

            def load_idx(i, b):
                r0 = (start + i) * _CH
                pltpu.sync_copy(dst_hbm.at[pl.ds(r0, _CH)], idx_d[b])

            def scatter(b):
                for j in range(_CH):
                    pltpu.sync_copy(ones_v, accum.at[idx_d[b].at[j]],
                                    add=True)

            load_idx(0, 0)

            def step(i, b):
                @pl.when(i + 1 < cnt)
                def _():
                    load_idx(i + 1, 1 - b)

                @pl.when(i < cnt)
                def _():
                    scatter(b)

            def pair(j, _):
                step(2 * j, 0)
                step(2 * j + 1, 1)
                return 0

            lax.fori_loop(0, (max_steps + 2) // 2, pair, 0)

        plsc.subcore_barrier()
        off = sid * rows_per_tile
        pltpu.sync_copy(accum.at[pl.ds(off, rows_per_tile)],
                        out_hbm.at[cid, pl.ds(off, rows_per_tile)])

    return pl.kernel(
        body,
        out_type=jax.ShapeDtypeStruct((_NC, n_pad, feat), jnp.float32),
        mesh=_mesh(),
        scratch_types=scratch,
        compiler_params=pltpu.CompilerParams(use_tc_tiling_on_sc=False),
    )


def _tc(body, grid, in_specs, out_specs, out_shape):
    return pl.pallas_call(
        body, grid=grid, in_specs=in_specs, out_specs=out_specs,
        out_shape=out_shape)


def kernel(x, edge_index, W1, b1, W2, b2, W3, b3):
    n = x.shape[0]
    e = edge_index.shape[1]
    src2d = edge_index[0].reshape(-1, 128)
    dst2d = edge_index[1].reshape(-1, 128)
    n_rows = e // 128
    n_pad = ((n + _NS * 8 - 1) // (_NS * 8)) * (_NS * 8)
    rows_per_tile = n_pad // _NS

    edge16 = _make_edge_pass(n, n_rows, 16, gather=True)
    deg_pass = _make_edge_pass(n, n_rows, 16, gather=False)

    zrow16 = jnp.zeros((rows_per_tile, 16), jnp.float32)
    ones16 = jnp.ones((128, 16), jnp.float32)

    bn = 2000
    grid = (n // bn,)
    part16 = pl.BlockSpec((_NC, bn, 16), lambda i: (0, i, 0))
    part1 = pl.BlockSpec((_NC, bn, 1), lambda i: (0, i, 0))
    vec_spec = pl.BlockSpec((bn, 1), lambda i: (i, 0))
    f16_spec = pl.BlockSpec((bn, 16), lambda i: (i, 0))

    def full(a):
        return pl.BlockSpec(a.shape, lambda i: tuple(0 for _ in a.shape))

    def deg_body(p_ref, dis_ref):
        p = p_ref[...]
        dis_ref[...] = lax.rsqrt(p[0, :, 0:1] + p[1, :, 0:1] + 1.0)

    def pre1_body(x_ref, w_ref, dis_ref, g_ref):
        t = jnp.dot(x_ref[...], w_ref[...],
                    preferred_element_type=jnp.float32)
        g_ref[...] = t * dis_ref[...]

    def mid1_body(p_ref, g1_ref, dis_ref, b1_ref, w2_ref, ga_ref, gb_ref):
        p = p_ref[...]
        dis = dis_ref[...]
        s = (p[0] + p[1] + g1_ref[...]) * dis + b1_ref[...]
        h = jnp.maximum(s, 0.0)
        t = jnp.dot(h, w2_ref[...], preferred_element_type=jnp.float32) * dis
        ga_ref[...] = t[:, :16]
        gb_ref[...] = t[:, 16:]

    def mid2_body(pa_ref, pb_ref, ga_ref, gb_ref, dis_ref, b2_ref, w3_ref,
                  g3_ref):
        pa = pa_ref[...]
        pb = pb_ref[...]
        dis = dis_ref[...]
        sa = pa[0] + pa[1] + ga_ref[...]
        sb = pb[0] + pb[1] + gb_ref[...]
        s = jnp.concatenate([sa, sb], axis=1) * dis + b2_ref[...]
        h = jnp.maximum(s, 0.0)
        t = jnp.dot(h, w3_ref[...], preferred_element_type=jnp.float32) * dis
        g3_ref[...] = jnp.concatenate(
            [t, jnp.zeros((t.shape[0], 15), jnp.float32)], axis=1)

    def fin_body(p_ref, g3_ref, dis_ref, b3_ref, out_ref):
        p = p_ref[...]
        s = p[0, :, 0:1] + p[1, :, 0:1] + g3_ref[...][:, 0:1]
        out_ref[...] = s * dis_ref[...] + b3_ref[...]

    b1r = b1.reshape(1, -1)
    b2r = b2.reshape(1, -1)
    b3r = b3.reshape(1, -1)

    deg_part = deg_pass(dst2d, ones16, zrow16)
    dis = _tc(deg_body, grid, [part16], vec_spec,
              jax.ShapeDtypeStruct((n, 1), jnp.float32))(deg_part)

    g1 = _tc(pre1_body, grid,
             [pl.BlockSpec((bn, 2), lambda i: (i, 0)), full(W1), vec_spec],
             f16_spec, jax.ShapeDtypeStruct((n, 16), jnp.float32))(
                 x, W1, dis)
    p1 = edge16(g1, src2d, dst2d, zrow16)

    g2a, g2b = _tc(mid1_body, grid,
                   [part16, f16_spec, vec_spec, full(b1r), full(W2)],
                   [f16_spec, f16_spec],
                   [jax.ShapeDtypeStruct((n, 16), jnp.float32)] * 2)(
                       p1, g1, dis, b1r, W2)
    p2a = edge16(g2a, src2d, dst2d, zrow16)
    p2b = edge16(g2b, src2d, dst2d, zrow16)

    g3 = _tc(mid2_body, grid,
             [part16, part16, f16_spec, f16_spec, vec_spec,
              full(b2r), full(W3)],
             f16_spec, jax.ShapeDtypeStruct((n, 16), jnp.float32))(
                 p2a, p2b, g2a, g2b, dis, b2r, W3)
    p3 = edge16(g3, src2d, dst2d, zrow16)

    out = _tc(fin_body, grid,
              [part16, f16_spec, vec_spec, full(b3r)],
              vec_spec, jax.ShapeDtypeStruct((n, 1), jnp.float32))(
                  p3, g3, dis, b3r)
    return out


# speedup vs baseline: 67.2536x; 1.1201x over previous
import functools

import jax
import jax.numpy as jnp
from jax import lax
from jax.experimental import pallas as pl
from jax.experimental.pallas import tpu as pltpu
from jax.experimental.pallas import tpu_sc as plsc

_NC = 2
_NS = 16
_NW = _NC * _NS
_CH = 5


def _mesh():
    return plsc.VectorSubcoreMesh(
        core_axis_name="c", subcore_axis_name="s",
        num_cores=_NC, num_subcores=_NS)


def _make_edge_pass(n_nodes, n_rows, feat, gather):
    n_chunks = n_rows // _CH
    base, rem = divmod(n_chunks, _NW)
    max_steps = base + (1 if rem else 0)
    n_pad = ((n_nodes + _NS * 8 - 1) // (_NS * 8)) * (_NS * 8)
    rows_per_tile = n_pad // _NS

    ce = _CH * 128
    scratch = {
        "idx_d0": pltpu.VMEM((ce,), jnp.int32),
        "idx_d1": pltpu.VMEM((ce,), jnp.int32),
        "accum": pltpu.VMEM_SHARED((n_pad, feat), jnp.float32),
    }
    if gather:
        scratch.update({
            "idx_s0": pltpu.VMEM((ce,), jnp.int32),
            "idx_s1": pltpu.VMEM((ce,), jnp.int32),
            "rows0": pltpu.VMEM((ce, feat), jnp.float32),
            "rows1": pltpu.VMEM((ce, feat), jnp.float32),
            "gsem0": pltpu.SemaphoreType.DMA,
            "gsem1": pltpu.SemaphoreType.DMA,
        })
    else:
        scratch["ones_v"] = pltpu.VMEM((ce, feat), jnp.float32)

    def body(*refs, **kw):
        if gather:
            table_hbm, src_hbm, dst_hbm, zrow_hbm, out_hbm = refs
        else:
            dst_hbm, ones_hbm, zrow_hbm, out_hbm = refs
        cid = lax.axis_index("c")
        sid = lax.axis_index("s")
        wid = sid * _NC + cid
        accum = kw["accum"]

        pltpu.sync_copy(zrow_hbm, accum.at[pl.ds(sid * rows_per_tile,
                                                 rows_per_tile)])
        if not gather:
            pltpu.sync_copy(ones_hbm, kw["ones_v"])
        plsc.subcore_barrier()

        cnt = base + jnp.where(wid < rem, 1, 0)
        start = wid * base + jnp.minimum(wid, rem)

        if gather:
            idx_s = (kw["idx_s0"], kw["idx_s1"])
            idx_d = (kw["idx_d0"], kw["idx_d1"])
            rows = (kw["rows0"], kw["rows1"])
            gsem = (kw["gsem0"], kw["gsem1"])

            def load_idx(i, b):
                e0 = (start + i) * ce
                pltpu.sync_copy(src_hbm.at[pl.ds(e0, ce)], idx_s[b])
                pltpu.sync_copy(dst_hbm.at[pl.ds(e0, ce)], idx_d[b])

            def fire(b):
                pltpu.async_copy(table_hbm.at[idx_s[b]], rows[b], gsem[b])

            def drain(b):
                pltpu.make_async_copy(table_hbm.at[idx_s[b]], rows[b],
                                      gsem[b]).wait()

            def scatter(b):
                pltpu.sync_copy(rows[b], accum.at[idx_d[b]], add=True)

            load_idx(0, 0)
            fire(0)
            load_idx(1, 1)

            def step(i, b):
                valid = i < cnt

                @pl.when(valid)
                def _():
                    drain(b)

                @pl.when(i + 1 < cnt)
                def _():
                    fire(1 - b)

                @pl.when(valid)
                def _():
                    scatter(b)

                @pl.when(i + 2 < cnt)
                def _():
                    load_idx(i + 2, b)

            def pair(j, _):
                step(2 * j, 0)
                step(2 * j + 1, 1)
                return 0

            lax.fori_loop(0, (max_steps + 2) // 2, pair, 0)
        else:
            idx_d = (kw["idx_d0"], kw["idx_d1"])
            ones_v = kw["ones_v"]

            def load_idx(i, b):
                e0 = (start + i) * ce
                pltpu.sync_copy(dst_hbm.at[pl.ds(e0, ce)], idx_d[b])

            def scatter(b):
                pltpu.sync_copy(ones_v, accum.at[idx_d[b]], add=True)

            load_idx(0, 0)

            def step(i, b):
                @pl.when(i + 1 < cnt)
                def _():
                    load_idx(i + 1, 1 - b)

                @pl.when(i < cnt)
                def _():
                    scatter(b)

            def pair(j, _):
                step(2 * j, 0)
                step(2 * j + 1, 1)
                return 0

            lax.fori_loop(0, (max_steps + 2) // 2, pair, 0)

        plsc.subcore_barrier()
        off = sid * rows_per_tile
        pltpu.sync_copy(accum.at[pl.ds(off, rows_per_tile)],
                        out_hbm.at[cid, pl.ds(off, rows_per_tile)])

    return pl.kernel(
        body,
        out_type=jax.ShapeDtypeStruct((_NC, n_pad, feat), jnp.float32),
        mesh=_mesh(),
        scratch_types=scratch,
        compiler_params=pltpu.CompilerParams(use_tc_tiling_on_sc=False),
    )


def _tc(body, grid, in_specs, out_specs, out_shape):
    return pl.pallas_call(
        body, grid=grid, in_specs=in_specs, out_specs=out_specs,
        out_shape=out_shape)


def kernel(x, edge_index, W1, b1, W2, b2, W3, b3):
    n = x.shape[0]
    e = edge_index.shape[1]
    src1d = edge_index[0]
    dst1d = edge_index[1]
    n_rows = e // 128
    n_pad = ((n + _NS * 8 - 1) // (_NS * 8)) * (_NS * 8)
    rows_per_tile = n_pad // _NS

    edge16 = _make_edge_pass(n, n_rows, 16, gather=True)
    deg_pass = _make_edge_pass(n, n_rows, 16, gather=False)

    zrow16 = jnp.zeros((rows_per_tile, 16), jnp.float32)
    ones16 = jnp.ones((_CH * 128, 16), jnp.float32)

    bn = 2000
    grid = (n // bn,)
    part16 = pl.BlockSpec((_NC, bn, 16), lambda i: (0, i, 0))
    part1 = pl.BlockSpec((_NC, bn, 1), lambda i: (0, i, 0))
    vec_spec = pl.BlockSpec((bn, 1), lambda i: (i, 0))
    f16_spec = pl.BlockSpec((bn, 16), lambda i: (i, 0))

    def full(a):
        return pl.BlockSpec(a.shape, lambda i: tuple(0 for _ in a.shape))

    def deg_body(p_ref, dis_ref):
        p = p_ref[...]
        dis_ref[...] = lax.rsqrt(p[0, :, 0:1] + p[1, :, 0:1] + 1.0)

    def pre1_body(x_ref, w_ref, dis_ref, g_ref):
        t = jnp.dot(x_ref[...], w_ref[...],
                    preferred_element_type=jnp.float32)
        g_ref[...] = t * dis_ref[...]

    def mid1_body(p_ref, g1_ref, dis_ref, b1_ref, w2_ref, ga_ref, gb_ref):
        p = p_ref[...]
        dis = dis_ref[...]
        s = (p[0] + p[1] + g1_ref[...]) * dis + b1_ref[...]
        h = jnp.maximum(s, 0.0)
        t = jnp.dot(h, w2_ref[...], preferred_element_type=jnp.float32) * dis
        ga_ref[...] = t[:, :16]
        gb_ref[...] = t[:, 16:]

    def mid2_body(pa_ref, pb_ref, ga_ref, gb_ref, dis_ref, b2_ref, w3_ref,
                  g3_ref):
        pa = pa_ref[...]
        pb = pb_ref[...]
        dis = dis_ref[...]
        sa = pa[0] + pa[1] + ga_ref[...]
        sb = pb[0] + pb[1] + gb_ref[...]
        s = jnp.concatenate([sa, sb], axis=1) * dis + b2_ref[...]
        h = jnp.maximum(s, 0.0)
        t = jnp.dot(h, w3_ref[...], preferred_element_type=jnp.float32) * dis
        g3_ref[...] = jnp.concatenate(
            [t, jnp.zeros((t.shape[0], 15), jnp.float32)], axis=1)

    def fin_body(p_ref, g3_ref, dis_ref, b3_ref, out_ref):
        p = p_ref[...]
        s = p[0, :, 0:1] + p[1, :, 0:1] + g3_ref[...][:, 0:1]
        out_ref[...] = s * dis_ref[...] + b3_ref[...]

    b1r = b1.reshape(1, -1)
    b2r = b2.reshape(1, -1)
    b3r = b3.reshape(1, -1)

    deg_part = deg_pass(dst1d, ones16, zrow16)
    dis = _tc(deg_body, grid, [part16], vec_spec,
              jax.ShapeDtypeStruct((n, 1), jnp.float32))(deg_part)

    g1 = _tc(pre1_body, grid,
             [pl.BlockSpec((bn, 2), lambda i: (i, 0)), full(W1), vec_spec],
             f16_spec, jax.ShapeDtypeStruct((n, 16), jnp.float32))(
                 x, W1, dis)
    p1 = edge16(g1, src1d, dst1d, zrow16)

    g2a, g2b = _tc(mid1_body, grid,
                   [part16, f16_spec, vec_spec, full(b1r), full(W2)],
                   [f16_spec, f16_spec],
                   [jax.ShapeDtypeStruct((n, 16), jnp.float32)] * 2)(
                       p1, g1, dis, b1r, W2)
    p2a = edge16(g2a, src1d, dst1d, zrow16)
    p2b = edge16(g2b, src1d, dst1d, zrow16)

    g3 = _tc(mid2_body, grid,
             [part16, part16, f16_spec, f16_spec, vec_spec,
              full(b2r), full(W3)],
             f16_spec, jax.ShapeDtypeStruct((n, 16), jnp.float32))(
                 p2a, p2b, g2a, g2b, dis, b2r, W3)
    p3 = edge16(g3, src1d, dst1d, zrow16)

    out = _tc(fin_body, grid,
              [part16, f16_spec, vec_spec, full(b3r)],
              vec_spec, jax.ShapeDtypeStruct((n, 1), jnp.float32))(
                  p3, g3, dis, b3r)
    return out


# speedup vs baseline: 75.9742x; 1.1297x over previous
import functools

import jax
import jax.numpy as jnp
from jax import lax
from jax.experimental import pallas as pl
from jax.experimental.pallas import tpu as pltpu
from jax.experimental.pallas import tpu_sc as plsc

_NC = 2
_NS = 16
_NW = _NC * _NS
_CE = 800


def _mesh():
    return plsc.VectorSubcoreMesh(
        core_axis_name="c", subcore_axis_name="s",
        num_cores=_NC, num_subcores=_NS)


def _n_pad(n_nodes):
    return ((n_nodes + _NS * 8 - 1) // (_NS * 8)) * (_NS * 8)


def _make_edge_pass(n_nodes, n_edges, feat, mode):
    n_chunks = n_edges // _CE
    workers = _NS if mode == "halves" else _NW
    base, rem = divmod(n_chunks, workers)
    max_steps = base + (1 if rem else 0)
    n_pad = _n_pad(n_nodes)
    rows_per_tile = n_pad // _NS
    gather = mode != "ones"

    scratch = {
        "idx_d0": pltpu.VMEM((_CE,), jnp.int32),
        "idx_d1": pltpu.VMEM((_CE,), jnp.int32),
        "accum": pltpu.VMEM_SHARED((n_pad, feat), jnp.float32),
    }
    if gather:
        scratch.update({
            "idx_s0": pltpu.VMEM((_CE,), jnp.int32),
            "idx_s1": pltpu.VMEM((_CE,), jnp.int32),
            "rows0": pltpu.VMEM((_CE, feat), jnp.float32),
            "rows1": pltpu.VMEM((_CE, feat), jnp.float32),
            "gsem0": pltpu.SemaphoreType.DMA,
            "gsem1": pltpu.SemaphoreType.DMA,
        })
    else:
        scratch["ones_v"] = pltpu.VMEM((_CE, feat), jnp.float32)

    def body(*refs, **kw):
        if gather:
            table_hbm, src_hbm, dst_hbm, zrow_hbm, out_hbm = refs
        else:
            dst_hbm, ones_hbm, zrow_hbm, out_hbm = refs
        cid = lax.axis_index("c")
        sid = lax.axis_index("s")
        wid = sid * _NC + cid if mode != "halves" else sid
        accum = kw["accum"]

        pltpu.sync_copy(zrow_hbm, accum.at[pl.ds(sid * rows_per_tile,
                                                 rows_per_tile)])
        if not gather:
            pltpu.sync_copy(ones_hbm, kw["ones_v"])
        plsc.subcore_barrier()

        cnt = base + jnp.where(wid < rem, 1, 0) if rem else base
        start = wid * base + (jnp.minimum(wid, rem) if rem else 0)

        if gather:
            idx_s = (kw["idx_s0"], kw["idx_s1"])
            idx_d = (kw["idx_d0"], kw["idx_d1"])
            rows = (kw["rows0"], kw["rows1"])
            gsem = (kw["gsem0"], kw["gsem1"])
            if mode == "halves":
                table = table_hbm.at[cid]
            else:
                table = table_hbm

            def load_idx(i, b):
                e0 = (start + i) * _CE
                pltpu.sync_copy(src_hbm.at[pl.ds(e0, _CE)], idx_s[b])
                pltpu.sync_copy(dst_hbm.at[pl.ds(e0, _CE)], idx_d[b])

            def fire(b):
                pltpu.async_copy(table.at[idx_s[b]], rows[b], gsem[b])

            def drain(b):
                pltpu.make_async_copy(table.at[idx_s[b]], rows[b],
                                      gsem[b]).wait()

            def scatter(b):
                pltpu.sync_copy(rows[b], accum.at[idx_d[b]], add=True)

            load_idx(0, 0)
            fire(0)
            load_idx(1, 1)

            def step(i, b):
                valid = i < cnt

                @pl.when(valid)
                def _():
                    drain(b)

                @pl.when(i + 1 < cnt)
                def _():
                    fire(1 - b)

                @pl.when(valid)
                def _():
                    scatter(b)

                @pl.when(i + 2 < cnt)
                def _():
                    load_idx(i + 2, b)

            def pair(j, _):
                step(2 * j, 0)
                step(2 * j + 1, 1)
                return 0

            lax.fori_loop(0, (max_steps + 2) // 2, pair, 0)
        else:
            idx_d = (kw["idx_d0"], kw["idx_d1"])
            ones_v = kw["ones_v"]

            def load_idx(i, b):
                e0 = (start + i) * _CE
                pltpu.sync_copy(dst_hbm.at[pl.ds(e0, _CE)], idx_d[b])

            def scatter(b):
                pltpu.sync_copy(ones_v, accum.at[idx_d[b]], add=True)

            load_idx(0, 0)

            def step(i, b):
                @pl.when(i + 1 < cnt)
                def _():
                    load_idx(i + 1, 1 - b)

                @pl.when(i < cnt)
                def _():
                    scatter(b)

            def pair(j, _):
                step(2 * j, 0)
                step(2 * j + 1, 1)
                return 0

            lax.fori_loop(0, (max_steps + 2) // 2, pair, 0)

        plsc.subcore_barrier()
        off = sid * rows_per_tile
        pltpu.sync_copy(accum.at[pl.ds(off, rows_per_tile)],
                        out_hbm.at[cid, pl.ds(off, rows_per_tile)])

    return pl.kernel(
        body,
        out_type=jax.ShapeDtypeStruct((_NC, n_pad, feat), jnp.float32),
        mesh=_mesh(),
        scratch_types=scratch,
        compiler_params=pltpu.CompilerParams(use_tc_tiling_on_sc=False),
    )


def _tc(body, grid, in_specs, out_specs, out_shape):
    return pl.pallas_call(
        body, grid=grid, in_specs=in_specs, out_specs=out_specs,
        out_shape=out_shape)


def kernel(x, edge_index, W1, b1, W2, b2, W3, b3):
    n = x.shape[0]
    e = edge_index.shape[1]
    src1d = edge_index[0]
    dst1d = edge_index[1]
    n_pad = _n_pad(n)
    rows_per_tile = n_pad // _NS

    edge16 = _make_edge_pass(n, e, 16, "split")
    edge16h = _make_edge_pass(n, e, 16, "halves")
    edge4 = _make_edge_pass(n, e, 16, "split")
    deg_pass = _make_edge_pass(n, e, 16, "ones")

    zrow16 = jnp.zeros((rows_per_tile, 16), jnp.float32)
    zrow4 = jnp.zeros((rows_per_tile, 16), jnp.float32)
    ones4 = jnp.ones((_CE, 16), jnp.float32)

    bn = 2000
    grid = (n // bn,)
    part16 = pl.BlockSpec((_NC, bn, 16), lambda i: (0, i, 0))
    part4 = pl.BlockSpec((_NC, bn, 16), lambda i: (0, i, 0))
    vec_spec = pl.BlockSpec((bn, 1), lambda i: (i, 0))
    f16_spec = pl.BlockSpec((bn, 16), lambda i: (i, 0))
    f4_spec = pl.BlockSpec((bn, 16), lambda i: (i, 0))
    half_spec = pl.BlockSpec((_NC, bn, 16), lambda i: (0, i, 0))

    def full(a):
        return pl.BlockSpec(a.shape, lambda i: tuple(0 for _ in a.shape))

    def pre1_body(p_ref, x_ref, w_ref, dis_ref, g_ref):
        p = p_ref[...]
        dis = lax.rsqrt(p[0, :, 0:1] + p[1, :, 0:1] + 1.0)
        dis_ref[...] = dis
        t = jnp.dot(x_ref[...], w_ref[...],
                    preferred_element_type=jnp.float32)
        g_ref[...] = t * dis

    def mid1_body(p_ref, g1_ref, dis_ref, b1_ref, w2_ref, g2_ref):
        p = p_ref[...]
        dis = dis_ref[...]
        s = (p[0] + p[1] + g1_ref[...]) * dis + b1_ref[...]
        h = jnp.maximum(s, 0.0)
        t = jnp.dot(h, w2_ref[...], preferred_element_type=jnp.float32) * dis
        g2_ref[0] = t[:, :16]
        g2_ref[1] = t[:, 16:]

    def mid2_body(p_ref, g2_ref, dis_ref, b2_ref, w3_ref, g3_ref):
        p = p_ref[...]
        g2 = g2_ref[...]
        dis = dis_ref[...]
        sa = p[0] + g2[0]
        sb = p[1] + g2[1]
        s = jnp.concatenate([sa, sb], axis=1) * dis + b2_ref[...]
        h = jnp.maximum(s, 0.0)
        t = jnp.dot(h, w3_ref[...], preferred_element_type=jnp.float32) * dis
        g3_ref[...] = jnp.concatenate(
            [t, jnp.zeros((t.shape[0], 15), jnp.float32)], axis=1)

    def fin_body(p_ref, g3_ref, dis_ref, b3_ref, out_ref):
        p = p_ref[...]
        s = p[0, :, 0:1] + p[1, :, 0:1] + g3_ref[...][:, 0:1]
        out_ref[...] = s * dis_ref[...] + b3_ref[...]

    b1r = b1.reshape(1, -1)
    b2r = b2.reshape(1, -1)
    b3r = b3.reshape(1, -1)

    deg_part = deg_pass(dst1d, ones4, zrow4)

    dis, g1 = _tc(pre1_body, grid,
                  [part4, pl.BlockSpec((bn, 2), lambda i: (i, 0)),
                   full(W1)],
                  [vec_spec, f16_spec],
                  [jax.ShapeDtypeStruct((n, 1), jnp.float32),
                   jax.ShapeDtypeStruct((n, 16), jnp.float32)])(
                      deg_part, x, W1)
    p1 = edge16(g1, src1d, dst1d, zrow16)

    g2 = _tc(mid1_body, grid,
             [part16, f16_spec, vec_spec, full(b1r), full(W2)],
             half_spec,
             jax.ShapeDtypeStruct((_NC, n, 16), jnp.float32))(
                 p1, g1, dis, b1r, W2)
    p2 = edge16h(g2, src1d, dst1d, zrow16)

    g3 = _tc(mid2_body, grid,
             [part16, half_spec, vec_spec, full(b2r), full(W3)],
             f4_spec, jax.ShapeDtypeStruct((n, 16), jnp.float32))(
                 p2, g2, dis, b2r, W3)
    p3 = edge4(g3, src1d, dst1d, zrow4)

    out = _tc(fin_body, grid,
              [part4, f4_spec, vec_spec, full(b3r)],
              vec_spec, jax.ShapeDtypeStruct((n, 1), jnp.float32))(
                  p3, g3, dis, b3r)
    return out


# speedup vs baseline: 75.9839x; 1.0001x over previous
import jax
import jax.numpy as jnp
from jax import lax
from jax.experimental import pallas as pl
from jax.experimental.pallas import tpu as pltpu
from jax.experimental.pallas import tpu_sc as plsc

_NC = 2
_NS = 16
_NW = _NC * _NS
_CE = 800


def _mesh():
    return plsc.VectorSubcoreMesh(
        core_axis_name="c", subcore_axis_name="s",
        num_cores=_NC, num_subcores=_NS)


def _n_pad(n_nodes):
    return ((n_nodes + _NS * 8 - 1) // (_NS * 8)) * (_NS * 8)


def _make_edge_pass(n_nodes, n_edges, feat, mode):
    n_chunks = n_edges // _CE
    workers = _NS if mode == "halves" else _NW
    base, rem = divmod(n_chunks, workers)
    max_steps = base + (1 if rem else 0)
    n_pad = _n_pad(n_nodes)
    rows_per_tile = n_pad // _NS
    gather = mode != "ones"

    scratch = {
        "idx_d0": pltpu.VMEM((_CE,), jnp.int32),
        "idx_d1": pltpu.VMEM((_CE,), jnp.int32),
        "accum": pltpu.VMEM_SHARED((n_pad, feat), jnp.float32),
    }
    if gather:
        scratch.update({
            "idx_s0": pltpu.VMEM((_CE,), jnp.int32),
            "idx_s1": pltpu.VMEM((_CE,), jnp.int32),
            "rows0": pltpu.VMEM((_CE, feat), jnp.float32),
            "rows1": pltpu.VMEM((_CE, feat), jnp.float32),
            "gsem0": pltpu.SemaphoreType.DMA,
            "gsem1": pltpu.SemaphoreType.DMA,
        })
    else:
        scratch["ones_v"] = pltpu.VMEM((_CE, feat), jnp.float32)

    def body(*refs, **kw):
        if gather:
            table_hbm, src_hbm, dst_hbm, zrow_hbm, out_hbm = refs
        else:
            dst_hbm, ones_hbm, zrow_hbm, out_hbm = refs
        cid = lax.axis_index("c")
        sid = lax.axis_index("s")
        wid = sid * _NC + cid if mode != "halves" else sid
        accum = kw["accum"]

        pltpu.sync_copy(zrow_hbm, accum.at[pl.ds(sid * rows_per_tile,
                                                 rows_per_tile)])
        if not gather:
            pltpu.sync_copy(ones_hbm, kw["ones_v"])
        plsc.subcore_barrier()

        cnt = base + jnp.where(wid < rem, 1, 0) if rem else base
        start = wid * base + (jnp.minimum(wid, rem) if rem else 0)

        if gather:
            idx_s = (kw["idx_s0"], kw["idx_s1"])
            idx_d = (kw["idx_d0"], kw["idx_d1"])
            rows = (kw["rows0"], kw["rows1"])
            gsem = (kw["gsem0"], kw["gsem1"])
            if mode == "halves":
                table = table_hbm.at[cid]
            else:
                table = table_hbm

            def load_idx(i, b):
                e0 = (start + i) * _CE
                pltpu.sync_copy(src_hbm.at[pl.ds(e0, _CE)], idx_s[b])
                pltpu.sync_copy(dst_hbm.at[pl.ds(e0, _CE)], idx_d[b])

            def fire(b):
                pltpu.async_copy(table.at[idx_s[b]], rows[b], gsem[b])

            def drain(b):
                pltpu.make_async_copy(table.at[idx_s[b]], rows[b],
                                      gsem[b]).wait()

            def scatter(b):
                pltpu.sync_copy(rows[b], accum.at[idx_d[b]], add=True)

            load_idx(0, 0)
            fire(0)
            load_idx(1, 1)

            def step(i, b):
                valid = i < cnt

                @pl.when(valid)
                def _():
                    drain(b)

                @pl.when(i + 1 < cnt)
                def _():
                    fire(1 - b)

                @pl.when(valid)
                def _():
                    scatter(b)

                @pl.when(i + 2 < cnt)
                def _():
                    load_idx(i + 2, b)

            def pair(j, _):
                step(2 * j, 0)
                step(2 * j + 1, 1)
                return 0

            lax.fori_loop(0, (max_steps + 2) // 2, pair, 0)
        else:
            idx_d = (kw["idx_d0"], kw["idx_d1"])
            ones_v = kw["ones_v"]

            def load_idx(i, b):
                e0 = (start + i) * _CE
                pltpu.sync_copy(dst_hbm.at[pl.ds(e0, _CE)], idx_d[b])

            def scatter(b):
                pltpu.sync_copy(ones_v, accum.at[idx_d[b]], add=True)

            load_idx(0, 0)

            def step(i, b):
                @pl.when(i + 1 < cnt)
                def _():
                    load_idx(i + 1, 1 - b)

                @pl.when(i < cnt)
                def _():
                    scatter(b)

            def pair(j, _):
                step(2 * j, 0)
                step(2 * j + 1, 1)
                return 0

            lax.fori_loop(0, (max_steps + 2) // 2, pair, 0)

        plsc.subcore_barrier()
        off = sid * rows_per_tile
        pltpu.sync_copy(accum.at[pl.ds(off, rows_per_tile)],
                        out_hbm.at[cid, pl.ds(off, rows_per_tile)])

    return pl.kernel(
        body,
        out_type=jax.ShapeDtypeStruct((_NC, n_pad, feat), jnp.float32),
        mesh=_mesh(),
        scratch_types=scratch,
        compiler_params=pltpu.CompilerParams(use_tc_tiling_on_sc=False),
    )


def _tc(body, grid, in_specs, out_specs, out_shape):
    return pl.pallas_call(
        body, grid=grid, in_specs=in_specs, out_specs=out_specs,
        out_shape=out_shape)


def kernel(x, edge_index, W1, b1, W2, b2, W3, b3):
    n = x.shape[0]
    e = edge_index.shape[1]
    src1d = edge_index[0]
    dst1d = edge_index[1]
    n_pad = _n_pad(n)
    rows_per_tile = n_pad // _NS

    edge16 = _make_edge_pass(n, e, 16, "split")
    edge16h = _make_edge_pass(n, e, 16, "halves")
    edge16b = _make_edge_pass(n, e, 16, "split")
    deg_pass = _make_edge_pass(n, e, 16, "ones")

    zrow16 = jnp.zeros((rows_per_tile, 16), jnp.float32)
    ones16 = jnp.ones((_CE, 16), jnp.float32)

    bn = 2000
    grid = (n // bn,)
    part16 = pl.BlockSpec((_NC, bn, 16), lambda i: (0, i, 0))
    vec_spec = pl.BlockSpec((bn, 1), lambda i: (i, 0))
    f16_spec = pl.BlockSpec((bn, 16), lambda i: (i, 0))
    f16b_spec = pl.BlockSpec((bn, 16), lambda i: (i, 0))
    half_spec = pl.BlockSpec((_NC, bn, 16), lambda i: (0, i, 0))

    def full(a):
        return pl.BlockSpec(a.shape, lambda i: tuple(0 for _ in a.shape))

    def pre1_body(p_ref, x_ref, w_ref, dis_ref, g_ref):
        p = p_ref[...]
        dis = lax.rsqrt(p[0, :, 0:1] + p[1, :, 0:1] + 1.0)
        dis_ref[...] = dis
        t = jnp.dot(x_ref[...], w_ref[...],
                    preferred_element_type=jnp.float32)
        g_ref[...] = t * dis

    def mid1_body(p_ref, g1_ref, dis_ref, b1_ref, w2_ref, g2_ref):
        p = p_ref[...]
        dis = dis_ref[...]
        s = (p[0] + p[1] + g1_ref[...]) * dis + b1_ref[...]
        h = jnp.maximum(s, 0.0)
        t = jnp.dot(h, w2_ref[...], preferred_element_type=jnp.float32) * dis
        g2_ref[0] = t[:, :16]
        g2_ref[1] = t[:, 16:]

    def mid2_body(p_ref, g2_ref, dis_ref, b2_ref, w3_ref, g3_ref):
        p = p_ref[...]
        g2 = g2_ref[...]
        dis = dis_ref[...]
        sa = p[0] + g2[0]
        sb = p[1] + g2[1]
        s = jnp.concatenate([sa, sb], axis=1) * dis + b2_ref[...]
        h = jnp.maximum(s, 0.0)
        t = jnp.dot(h, w3_ref[...], preferred_element_type=jnp.float32) * dis
        g3_ref[...] = jnp.concatenate(
            [t, jnp.zeros((t.shape[0], 15), jnp.float32)], axis=1)

    def fin_body(p_ref, g3_ref, dis_ref, b3_ref, out_ref):
        p = p_ref[...]
        s = p[0, :, 0:1] + p[1, :, 0:1] + g3_ref[...][:, 0:1]
        out_ref[...] = s * dis_ref[...] + b3_ref[...]

    b1r = b1.reshape(1, -1)
    b2r = b2.reshape(1, -1)
    b3r = b3.reshape(1, -1)

    deg_part = deg_pass(dst1d, ones16, zrow16)

    dis, g1 = _tc(pre1_body, grid,
                  [part16, pl.BlockSpec((bn, 2), lambda i: (i, 0)),
                   full(W1)],
                  [vec_spec, f16_spec],
                  [jax.ShapeDtypeStruct((n, 1), jnp.float32),
                   jax.ShapeDtypeStruct((n, 16), jnp.float32)])(
                      deg_part, x, W1)
    p1 = edge16(g1, src1d, dst1d, zrow16)

    g2 = _tc(mid1_body, grid,
             [part16, f16_spec, vec_spec, full(b1r), full(W2)],
             half_spec,
             jax.ShapeDtypeStruct((_NC, n, 16), jnp.float32))(
                 p1, g1, dis, b1r, W2)
    p2 = edge16h(g2, src1d, dst1d, zrow16)

    g3 = _tc(mid2_body, grid,
             [part16, half_spec, vec_spec, full(b2r), full(W3)],
             f16b_spec, jax.ShapeDtypeStruct((n, 16), jnp.float32))(
                 p2, g2, dis, b2r, W3)
    p3 = edge16b(g3, src1d, dst1d, zrow16)

    out = _tc(fin_body, grid,
              [part16, f16b_spec, vec_spec, full(b3r)],
              vec_spec, jax.ShapeDtypeStruct((n, 1), jnp.float32))(
                  p3, g3, dis, b3r)
    return out
